# baseline (speedup 1.0000x reference)
import jax
import jax.numpy as jnp
from jax.experimental import pallas as pl
from jax.experimental.pallas import tpu as pltpu

H = 128
W = 128
CIN = 256
CMID = 512
NSC = 6
NBB = 12
BH = 8
M = BH * W
NBLK = H // BH


def _fused_rpn_body(x_ref, w1_ref, w2_ref, w3_ref, b1_ref, b2_ref, b3_ref,
                    score_ref, bbox_ref, xcat, w1s, w23s):
    i = pl.program_id(0)

    @pl.when(i == 0)
    def _prep():
        zrow = jnp.zeros((1, W, 3 * CIN), jnp.bfloat16)
        xcat[0:1] = zrow
        xcat[H + 1:H + 2] = zrow
        xcat[:, 0:1, 0:CIN] = jnp.zeros((H + 2, 1, CIN), jnp.bfloat16)
        xcat[:, W - 1:W, 2 * CIN:3 * CIN] = jnp.zeros((H + 2, 1, CIN),
                                                      jnp.bfloat16)
        w1s[...] = w1_ref[...].astype(jnp.bfloat16).reshape(3, 3 * CIN, CMID)
        w23s[:, :NSC] = w2_ref[0, 0].astype(jnp.bfloat16)
        w23s[:, NSC:NSC + NBB] = w3_ref[0, 0].astype(jnp.bfloat16)

    @pl.when(i < NBLK)
    def _convert():
        xb = x_ref[0].astype(jnp.bfloat16)
        r0 = i * BH + 1
        xcat[pl.ds(r0, BH), 1:W, 0:CIN] = xb[:, 0:W - 1, :]
        xcat[pl.ds(r0, BH), :, CIN:2 * CIN] = xb
        xcat[pl.ds(r0, BH), 0:W - 1, 2 * CIN:3 * CIN] = xb[:, 1:W, :]

    @pl.when(i > 0)
    def _block():
        j = i - 1
        acc = jnp.zeros((M, CMID), jnp.float32)
        for kh in range(3):
            op = xcat[pl.ds(j * BH + kh, BH)].reshape(M, 3 * CIN)
            acc = acc + jnp.dot(op, w1s[kh],
                                preferred_element_type=jnp.float32)
        y = jnp.maximum(acc + b1_ref[...], 0.0).astype(jnp.bfloat16)
        heads = jnp.dot(y, w23s[...], preferred_element_type=jnp.float32)
        score_ref[...] = heads[:, :NSC] + b2_ref[...]
        for a in range(3):
            bbox_ref[pl.Slice(a, M, 3), :] = (
                heads[:, NSC + 4 * a:NSC + 4 * (a + 1)]
                + b3_ref[4 * a:4 * (a + 1)])


def kernel(inputs, W1, b1, W2, b2, W3, b3):
    score, bbox = pl.pallas_call(
        _fused_rpn_body,
        grid=(NBLK + 1,),
        in_specs=[
            pl.BlockSpec((1, BH, W, CIN),
                         lambda i: (0, jnp.minimum(i, NBLK - 1), 0, 0)),
            pl.BlockSpec((3, 3, CIN, CMID), lambda i: (0, 0, 0, 0)),
            pl.BlockSpec((1, 1, CMID, NSC), lambda i: (0, 0, 0, 0)),
            pl.BlockSpec((1, 1, CMID, NBB), lambda i: (0, 0, 0, 0)),
            pl.BlockSpec((CMID,), lambda i: (0,)),
            pl.BlockSpec((NSC,), lambda i: (0,)),
            pl.BlockSpec((NBB,), lambda i: (0,)),
        ],
        out_specs=[
            pl.BlockSpec((M, NSC), lambda i: (jnp.maximum(i - 1, 0), 0)),
            pl.BlockSpec((3 * M, 4), lambda i: (jnp.maximum(i - 1, 0), 0)),
        ],
        out_shape=[
            jax.ShapeDtypeStruct((H * W, NSC), jnp.float32),
            jax.ShapeDtypeStruct((3 * H * W, 4), jnp.float32),
        ],
        scratch_shapes=[
            pltpu.VMEM((H + 2, W, 3 * CIN), jnp.bfloat16),
            pltpu.VMEM((3, 3 * CIN, CMID), jnp.bfloat16),
            pltpu.VMEM((CMID, NSC + NBB), jnp.bfloat16),
        ],
    )(inputs, W1, W2, W3, b1, b2, b3)

    return score, bbox


# speedup vs baseline: 1.0324x; 1.0324x over previous
import jax
import jax.numpy as jnp
from jax.experimental import pallas as pl
from jax.experimental.pallas import tpu as pltpu

H = 128
W = 128
CIN = 256
CMID = 512
NSC = 6
NBB = 12
BH = 16
M = BH * W
NBLK = H // BH


def _fused_rpn_body(x_ref, w1_ref, w2_ref, w3_ref, b1_ref, b2_ref, b3_ref,
                    score_ref, bbox_ref, xcat, w1s, w23s):
    i = pl.program_id(0)

    @pl.when(i == 0)
    def _prep():
        zrow = jnp.zeros((1, W, 3 * CIN), jnp.bfloat16)
        xcat[0:1] = zrow
        xcat[H + 1:H + 2] = zrow
        xcat[:, 0:1, 0:CIN] = jnp.zeros((H + 2, 1, CIN), jnp.bfloat16)
        xcat[:, W - 1:W, 2 * CIN:3 * CIN] = jnp.zeros((H + 2, 1, CIN),
                                                      jnp.bfloat16)
        w1s[...] = w1_ref[...].astype(jnp.bfloat16).reshape(3, 3 * CIN, CMID)
        w23s[:, :NSC] = w2_ref[0, 0].astype(jnp.bfloat16)
        w23s[:, NSC:NSC + NBB] = w3_ref[0, 0].astype(jnp.bfloat16)

    @pl.when(i < NBLK)
    def _convert():
        xb = x_ref[0].astype(jnp.bfloat16)
        r0 = i * BH + 1
        xcat[pl.ds(r0, BH), 1:W, 0:CIN] = xb[:, 0:W - 1, :]
        xcat[pl.ds(r0, BH), :, CIN:2 * CIN] = xb
        xcat[pl.ds(r0, BH), 0:W - 1, 2 * CIN:3 * CIN] = xb[:, 1:W, :]

    @pl.when(i > 0)
    def _block():
        j = i - 1
        acc = jnp.zeros((M, CMID), jnp.float32)
        for kh in range(3):
            op = xcat[pl.ds(j * BH + kh, BH)].reshape(M, 3 * CIN)
            acc = acc + jnp.dot(op, w1s[kh],
                                preferred_element_type=jnp.float32)
        y = jnp.maximum(acc + b1_ref[...], 0.0).astype(jnp.bfloat16)
        heads = jnp.dot(y, w23s[...], preferred_element_type=jnp.float32)
        score_ref[...] = heads[:, :NSC] + b2_ref[...]
        for a in range(3):
            bbox_ref[pl.Slice(a, M, 3), :] = (
                heads[:, NSC + 4 * a:NSC + 4 * (a + 1)]
                + b3_ref[4 * a:4 * (a + 1)])


def kernel(inputs, W1, b1, W2, b2, W3, b3):
    score, bbox = pl.pallas_call(
        _fused_rpn_body,
        grid=(NBLK + 1,),
        in_specs=[
            pl.BlockSpec((1, BH, W, CIN),
                         lambda i: (0, jnp.minimum(i, NBLK - 1), 0, 0)),
            pl.BlockSpec((3, 3, CIN, CMID), lambda i: (0, 0, 0, 0)),
            pl.BlockSpec((1, 1, CMID, NSC), lambda i: (0, 0, 0, 0)),
            pl.BlockSpec((1, 1, CMID, NBB), lambda i: (0, 0, 0, 0)),
            pl.BlockSpec((CMID,), lambda i: (0,)),
            pl.BlockSpec((NSC,), lambda i: (0,)),
            pl.BlockSpec((NBB,), lambda i: (0,)),
        ],
        out_specs=[
            pl.BlockSpec((M, NSC), lambda i: (jnp.maximum(i - 1, 0), 0)),
            pl.BlockSpec((3 * M, 4), lambda i: (jnp.maximum(i - 1, 0), 0)),
        ],
        out_shape=[
            jax.ShapeDtypeStruct((H * W, NSC), jnp.float32),
            jax.ShapeDtypeStruct((3 * H * W, 4), jnp.float32),
        ],
        scratch_shapes=[
            pltpu.VMEM((H + 2, W, 3 * CIN), jnp.bfloat16),
            pltpu.VMEM((3, 3 * CIN, CMID), jnp.bfloat16),
            pltpu.VMEM((CMID, NSC + NBB), jnp.bfloat16),
        ],
    )(inputs, W1, W2, W3, b1, b2, b3)

    return score, bbox
